# Initial kernel scaffold; baseline (speedup 1.0000x reference)
import jax
import jax.numpy as jnp
from jax.experimental import pallas as pl


def kernel(feature_indices, feature_values, weight, bias):
    raise NotImplementedError("write your pallas kernel here")


# speedup vs baseline: 4.8843x; 4.8843x over previous
import functools

import jax
import jax.numpy as jnp
from jax import lax
from jax.experimental import pallas as pl
from jax.experimental.pallas import tpu as pltpu
from jax.experimental.pallas import tpu_sc as plsc

B = 16384
K = 32
O = 256
NC = 2
NS = 16
NW = NC * NS
BPW = B // NW
GROUP = 4
GK = GROUP * K
NG = BPW // GROUP
GPC = 16
L = 16
NJ = O // L


def _sc_body(fi, fv, w, bias, out, idx_v, vals_v, bias_v, rows0, rows1,
             out_v, sem0, sem1):
    c = lax.axis_index("c")
    s = lax.axis_index("s")
    wid = s * NC + c

    pltpu.sync_copy(fi.at[wid], idx_v)
    pltpu.sync_copy(fv.at[wid], vals_v)
    pltpu.sync_copy(bias, bias_v)

    pltpu.async_copy(w.at[idx_v.at[0]], rows0, sem0)
    pltpu.async_copy(w.at[idx_v.at[1]], rows1, sem1)

    def group_iter(i, carry):
        for b, (rows, sem) in enumerate(((rows0, sem0), (rows1, sem1))):
            gi = 2 * i + b
            pltpu.make_async_copy(w.at[idx_v.at[gi]], rows, sem).wait()

            for r in range(GROUP):
                vbase = gi * GK + r * K
                vrows = [vals_v[pl.ds(vbase + h * L, L)] for h in range(K // L)]
                vb = [
                    jnp.full((L,), vrows[k // L][k % L], jnp.float32)
                    for k in range(K)
                ]
                orow = (gi % GPC) * GROUP + r

                def j_iter(j, _, vb=vb, orow=orow, rows=rows, r=r):
                    acc = bias_v[pl.ds(j * L, L)]
                    for k in range(K):
                        acc = acc + rows[r * K + k, pl.ds(j * L, L)] * vb[k]
                    out_v[orow, pl.ds(j * L, L)] = acc
                    return _

                lax.fori_loop(0, NJ, j_iter, 0)

            @pl.when(gi + 2 < NG)
            def _prefetch(rows=rows, sem=sem, gi=gi):
                pltpu.async_copy(w.at[idx_v.at[gi + 2]], rows, sem)

            @pl.when(gi % GPC == GPC - 1)
            def _flush(gi=gi):
                base = pl.multiple_of(wid * BPW + (gi - (GPC - 1)) * GROUP,
                                      GPC * GROUP)
                pltpu.sync_copy(out_v, out.at[pl.ds(base, GPC * GROUP)])
        return carry

    lax.fori_loop(0, NG // 2, group_iter, 0)


def kernel(feature_indices, feature_values, weight, bias):
    fi = feature_indices.reshape(NW, NG, GK)
    fv = feature_values.reshape(NW, NG * GK)

    mesh = plsc.VectorSubcoreMesh(core_axis_name="c", subcore_axis_name="s")
    run = pl.kernel(
        _sc_body,
        out_type=jax.ShapeDtypeStruct((B, O), jnp.float32),
        mesh=mesh,
        scratch_types=[
            pltpu.VMEM((NG, GK), jnp.int32),
            pltpu.VMEM((NG * GK,), jnp.float32),
            pltpu.VMEM((O,), jnp.float32),
            pltpu.VMEM((GK, O), jnp.float32),
            pltpu.VMEM((GK, O), jnp.float32),
            pltpu.VMEM((GPC * GROUP, O), jnp.float32),
            pltpu.SemaphoreType.DMA,
            pltpu.SemaphoreType.DMA,
        ],
    )
    return run(fi, fv, weight, bias)
